# baseline (speedup 1.0000x reference)
import jax
import jax.numpy as jnp
from jax.experimental import pallas as pl

NUM_EXPERTS = 16
HIDDEN = 64
FLAT = NUM_EXPERTS * HIDDEN
GPAD = 128


def _moe_body(x_ref, wcat_ref, bcat_ref, gw2_ref, gb2_ref,
              ew2_ref, eb2_ref, out_ref):
    f = x_ref.shape[1]
    xb16 = x_ref[0].astype(jnp.bfloat16)

    ha = jnp.maximum(
        jnp.dot(xb16, wcat_ref[...], preferred_element_type=jnp.float32)
        + bcat_ref[...], 0.0)
    h = ha[:, :FLAT]
    g1 = ha[:, FLAT:]

    logits = (jax.lax.dot_general(g1, gw2_ref[...], (((1,), (1,)), ((), ())),
                                  preferred_element_type=jnp.float32)
              + gb2_ref[...])
    m = jnp.max(logits, axis=1, keepdims=True)
    el = jnp.exp(logits - m)
    z = jnp.sum(el, axis=1, keepdims=True)

    m1 = jnp.max(el, axis=1, keepdims=True)
    el2 = jnp.where(el == m1, -1.0, el)
    m2 = jnp.max(el2, axis=1, keepdims=True)
    w = jnp.where(el >= m2, el, 0.0) / z

    c = jax.lax.dot_general(w, h, (((0,), (0,)), ((), ())),
                            preferred_element_type=jnp.float32)
    eidx = jax.lax.broadcasted_iota(jnp.int32, (NUM_EXPERTS, FLAT), 0)
    jidx = jax.lax.broadcasted_iota(jnp.int32, (NUM_EXPERTS, FLAT), 1)
    s = jnp.sum(jnp.where(jidx // HIDDEN == eidx, c, 0.0),
                axis=0, keepdims=True)
    wsum = jnp.sum(w, axis=0, keepdims=True)

    acc = jax.lax.dot_general(wsum, eb2_ref[...], (((1,), (0,)), ((), ())),
                              preferred_element_type=jnp.float32)
    for e in range(NUM_EXPERTS):
        acc = acc + jax.lax.dot_general(
            s[:, e * HIDDEN:(e + 1) * HIDDEN], ew2_ref[e],
            (((1,), (1,)), ((), ())),
            preferred_element_type=jnp.float32)
    out_ref[...] = (acc * (1.0 / f))[None]


def kernel(x, gw1, gb1, gw2, gb2, ew1, eb1, ew2, eb2):
    B, F, D = x.shape
    E, H, _ = ew1.shape
    O = ew2.shape[1]

    w1t = ew1.reshape(E * H, D).T.astype(jnp.bfloat16)
    gw1t = gw1.T.astype(jnp.bfloat16)
    wcat = jnp.concatenate(
        [w1t, gw1t, jnp.zeros((D, GPAD - H), jnp.bfloat16)], axis=1)
    bcat = jnp.concatenate(
        [eb1.reshape(1, E * H), gb1.reshape(1, H),
         jnp.zeros((1, GPAD - H), jnp.float32)], axis=1)
    gw2p = jnp.concatenate(
        [gw2, jnp.zeros((E, GPAD - H), gw2.dtype)], axis=1)
    gb2r = gb2.reshape(1, E)

    full = lambda *shape: pl.BlockSpec(shape, lambda b: (0,) * len(shape))
    out = pl.pallas_call(
        _moe_body,
        grid=(B,),
        in_specs=[
            pl.BlockSpec((1, F, D), lambda b: (b, 0, 0)),
            full(D, FLAT + GPAD), full(1, FLAT + GPAD),
            full(E, GPAD), full(1, E),
            full(E, O, H), full(E, O),
        ],
        out_specs=pl.BlockSpec((1, 1, O), lambda b: (b, 0, 0)),
        out_shape=jax.ShapeDtypeStruct((B, 1, O), x.dtype),
    )(x, wcat, bcat, gw2p, gb2r, ew2, eb2)
    return out.reshape(B, 1, 1, O)


# speedup vs baseline: 1.0150x; 1.0150x over previous
import functools
import jax
import jax.numpy as jnp
from jax import lax
from jax.experimental import pallas as pl
from jax.experimental.pallas import tpu as pltpu
from jax.experimental.pallas import tpu_sc as plsc

NUM_EXPERTS = 16
HIDDEN = 64
FLAT = NUM_EXPERTS * HIDDEN


def _gate_body(x_ref, gw1t_ref, gb1_ref, gw2t_ref, gb2_ref, logits_ref):
    xb16 = x_ref[...].astype(jnp.bfloat16)
    g1 = jnp.maximum(
        jnp.dot(xb16, gw1t_ref[...], preferred_element_type=jnp.float32)
        + gb1_ref[...], 0.0)
    logits_ref[...] = (
        jnp.dot(g1, gw2t_ref[...], preferred_element_type=jnp.float32)
        + gb2_ref[...])


def _expert_body(x_ref, w_ref, w1t_ref, b1_ref, w2_ref, eb2_ref,
                 emat_ref, out_ref):
    xb16 = x_ref[...].astype(jnp.bfloat16)
    f = xb16.shape[0]
    h = jnp.maximum(
        jnp.dot(xb16, w1t_ref[...], preferred_element_type=jnp.float32)
        + b1_ref[...], 0.0)
    w = w_ref[...]
    c = jax.lax.dot_general(w, h, (((0,), (0,)), ((), ())),
                            preferred_element_type=jnp.float32)
    s = jnp.sum(c * emat_ref[...], axis=0, keepdims=True)
    wsum = jnp.sum(w, axis=0, keepdims=True)
    out = (jnp.dot(s.astype(jnp.bfloat16), w2_ref[...],
                   preferred_element_type=jnp.float32)
           + jnp.dot(wsum, eb2_ref[...], preferred_element_type=jnp.float32))
    out_ref[...] = (out * (1.0 / f))[None]


def _make_sc_router(T, rows_per_worker):
    mesh = plsc.VectorSubcoreMesh(core_axis_name="c", subcore_axis_name="s")
    info = plsc.get_sparse_core_info()
    num_cores = info.num_cores

    @functools.partial(
        pl.kernel, mesh=mesh,
        compiler_params=pltpu.CompilerParams(needs_layout_passes=False),
        out_type=jax.ShapeDtypeStruct((T, NUM_EXPERTS), jnp.float32),
        scratch_types=[
            pltpu.VMEM((rows_per_worker, NUM_EXPERTS), jnp.float32),
            pltpu.VMEM((rows_per_worker, NUM_EXPERTS), jnp.float32),
        ],
    )
    def route(logits_hbm, w_hbm, lbuf, wbuf):
        wid = lax.axis_index("s") * num_cores + lax.axis_index("c")
        base = wid * rows_per_worker
        pltpu.sync_copy(logits_hbm.at[pl.ds(base, rows_per_worker)], lbuf)

        def body(i, carry):
            lv = lbuf[i]
            m = jnp.max(lv)
            el = jnp.exp(lv - m)
            z = jnp.sum(el)
            m1 = jnp.max(el)
            el2 = jnp.where(el == m1, -1.0, el)
            m2 = jnp.max(el2)
            wbuf[i] = jnp.where(el >= m2, el, 0.0) / z
            return carry

        lax.fori_loop(0, rows_per_worker, body, 0)
        pltpu.sync_copy(wbuf, w_hbm.at[pl.ds(base, rows_per_worker)])

    return route


def kernel(x, gw1, gb1, gw2, gb2, ew1, eb1, ew2, eb2):
    B, F, D = x.shape
    E, H, _ = ew1.shape
    O = ew2.shape[1]
    T = B * F

    xf = x.reshape(T, D)
    gw1t = gw1.T.astype(jnp.bfloat16)
    gw2t = gw2.T
    gb2r = gb2.reshape(1, E)
    w1t = ew1.reshape(E * H, D).T.astype(jnp.bfloat16)
    b1r = eb1.reshape(1, E * H)
    w2 = ew2.transpose(0, 2, 1).reshape(E * H, O).astype(jnp.bfloat16)
    emat = jnp.kron(jnp.eye(E, dtype=x.dtype), jnp.ones((1, H), dtype=x.dtype))

    full = lambda *shape: pl.BlockSpec(shape, lambda b: (0,) * len(shape))

    logits = pl.pallas_call(
        _gate_body,
        grid=(B,),
        in_specs=[
            pl.BlockSpec((F, D), lambda b: (b, 0)),
            full(D, H), full(1, H), full(H, E), full(1, E),
        ],
        out_specs=pl.BlockSpec((F, E), lambda b: (b, 0)),
        out_shape=jax.ShapeDtypeStruct((T, E), jnp.float32),
    )(xf, gw1t, gb1.reshape(1, H), gw2t, gb2r)

    info = plsc.get_sparse_core_info()
    num_workers = info.num_cores * info.num_subcores
    w = _make_sc_router(T, T // num_workers)(logits)

    out = pl.pallas_call(
        _expert_body,
        grid=(B,),
        in_specs=[
            pl.BlockSpec((F, D), lambda b: (b, 0)),
            pl.BlockSpec((F, E), lambda b: (b, 0)),
            full(D, E * H), full(1, E * H), full(E * H, O), full(E, O),
            full(E, E * H),
        ],
        out_specs=pl.BlockSpec((1, 1, O), lambda b: (b, 0, 0)),
        out_shape=jax.ShapeDtypeStruct((B, 1, O), x.dtype),
    )(xf, w, w1t, b1r, w2, eb2, emat)
    return out.reshape(B, 1, 1, O)


# speedup vs baseline: 1.0300x; 1.0148x over previous
import functools
import jax
import jax.numpy as jnp
from jax import lax
from jax.experimental import pallas as pl
from jax.experimental.pallas import tpu as pltpu
from jax.experimental.pallas import tpu_sc as plsc

NUM_EXPERTS = 16
HIDDEN = 64
FLAT = NUM_EXPERTS * HIDDEN


def _pass1_body(x_ref, w1cat_ref, b1cat_ref, gw2t_ref, gb2_ref,
                h_ref, logits_ref):
    xb16 = x_ref[...].astype(jnp.bfloat16)
    acc = jnp.maximum(
        jnp.dot(xb16, w1cat_ref[...], preferred_element_type=jnp.float32)
        + b1cat_ref[...], 0.0)
    h_ref[...] = acc[:, :FLAT].astype(jnp.bfloat16)
    g1 = acc[:, FLAT:]
    logits_ref[...] = (
        jnp.dot(g1, gw2t_ref[...], preferred_element_type=jnp.float32)
        + gb2_ref[...])


def _pass2_body(h_ref, w_ref, w2_ref, eb2_ref, emat_ref, out_ref):
    w = w_ref[...]
    f = w.shape[0]
    c = jax.lax.dot_general(w.astype(jnp.bfloat16), h_ref[...],
                            (((0,), (0,)), ((), ())),
                            preferred_element_type=jnp.float32)
    s = jnp.sum(c * emat_ref[...], axis=0, keepdims=True)
    wsum = jnp.sum(w, axis=0, keepdims=True)
    out = (jnp.dot(s.astype(jnp.bfloat16), w2_ref[...],
                   preferred_element_type=jnp.float32)
           + jnp.dot(wsum, eb2_ref[...], preferred_element_type=jnp.float32))
    out_ref[...] = (out * (1.0 / f))[None]


def _make_sc_router(T, rows_per_worker):
    mesh = plsc.VectorSubcoreMesh(core_axis_name="c", subcore_axis_name="s")
    info = plsc.get_sparse_core_info()
    num_cores = info.num_cores

    @functools.partial(
        pl.kernel, mesh=mesh,
        compiler_params=pltpu.CompilerParams(needs_layout_passes=False),
        out_type=jax.ShapeDtypeStruct((T, NUM_EXPERTS), jnp.float32),
        scratch_types=[
            pltpu.VMEM((rows_per_worker, NUM_EXPERTS), jnp.float32),
            pltpu.VMEM((rows_per_worker, NUM_EXPERTS), jnp.float32),
        ],
    )
    def route(logits_hbm, w_hbm, lbuf, wbuf):
        wid = lax.axis_index("s") * num_cores + lax.axis_index("c")
        base = wid * rows_per_worker
        pltpu.sync_copy(logits_hbm.at[pl.ds(base, rows_per_worker)], lbuf)

        def body(i, carry):
            lv = lbuf[i]
            m = jnp.max(lv)
            el = jnp.exp(lv - m)
            z = jnp.sum(el)
            m1 = jnp.max(el)
            el2 = jnp.where(el == m1, -1.0, el)
            m2 = jnp.max(el2)
            wbuf[i] = jnp.where(el >= m2, el, 0.0) / z
            return carry

        lax.fori_loop(0, rows_per_worker, body, 0)
        pltpu.sync_copy(wbuf, w_hbm.at[pl.ds(base, rows_per_worker)])

    return route


def kernel(x, gw1, gb1, gw2, gb2, ew1, eb1, ew2, eb2):
    B, F, D = x.shape
    E, H, _ = ew1.shape
    O = ew2.shape[1]
    T = B * F

    xf = x.reshape(T, D)
    w1t = ew1.reshape(E * H, D).T.astype(jnp.bfloat16)
    gw1t = gw1.T.astype(jnp.bfloat16)
    w1cat = jnp.concatenate([w1t, gw1t], axis=1)
    b1cat = jnp.concatenate(
        [eb1.reshape(1, E * H), gb1.reshape(1, H)], axis=1)
    gw2t = gw2.T
    gb2r = gb2.reshape(1, E)
    w2 = ew2.transpose(0, 2, 1).reshape(E * H, O).astype(jnp.bfloat16)
    emat = jnp.kron(jnp.eye(E, dtype=x.dtype), jnp.ones((1, H), dtype=x.dtype))

    full = lambda *shape: pl.BlockSpec(shape, lambda b: (0,) * len(shape))

    h, logits = pl.pallas_call(
        _pass1_body,
        grid=(B,),
        in_specs=[
            pl.BlockSpec((F, D), lambda b: (b, 0)),
            full(D, E * H + H), full(1, E * H + H), full(H, E), full(1, E),
        ],
        out_specs=[
            pl.BlockSpec((F, E * H), lambda b: (b, 0)),
            pl.BlockSpec((F, E), lambda b: (b, 0)),
        ],
        out_shape=[
            jax.ShapeDtypeStruct((T, E * H), jnp.bfloat16),
            jax.ShapeDtypeStruct((T, E), jnp.float32),
        ],
    )(xf, w1cat, b1cat, gw2t, gb2r)

    info = plsc.get_sparse_core_info()
    num_workers = info.num_cores * info.num_subcores
    w = _make_sc_router(T, T // num_workers)(logits)

    out = pl.pallas_call(
        _pass2_body,
        grid=(B,),
        in_specs=[
            pl.BlockSpec((F, E * H), lambda b: (b, 0)),
            pl.BlockSpec((F, E), lambda b: (b, 0)),
            full(E * H, O), full(E, O), full(E, E * H),
        ],
        out_specs=pl.BlockSpec((1, 1, O), lambda b: (b, 0, 0)),
        out_shape=jax.ShapeDtypeStruct((B, 1, O), x.dtype),
    )(h, w, w2, eb2, emat)
    return out.reshape(B, 1, 1, O)
